# Initial kernel scaffold; baseline (speedup 1.0000x reference)
import jax
import jax.numpy as jnp
from jax.experimental import pallas as pl


def kernel(node_x, eb_nodes_map, eb_batch, nb_edges_map, nb_batch, W1, b1, W2, b2):
    raise NotImplementedError("write your pallas kernel here")


# speedup vs baseline: 4.8260x; 4.8260x over previous
import functools

import jax
import jax.numpy as jnp
from jax import lax
from jax.experimental import pallas as pl
from jax.experimental.pallas import tpu as pltpu
from jax.experimental.pallas import tpu_sc as plsc

N_NODES = 10000
N_EDGES = 20000
N_MEMB = 320000
D_FEAT = 128

NC = 2
NS = 16
L = 16

NP = 10240
EP = 20480
BATCH = 128
NBATCH = 160
MP = NS * NBATCH * BATCH
NHALF = NBATCH // 2
C = 64
SLAB = 128

_MESH = plsc.VectorSubcoreMesh(core_axis_name="c", subcore_axis_name="s")


def _counts_body(segs, recip_e, recip_n, sh_cnt, ids_v, ones_v, cb):
  cid = lax.axis_index("c")
  sid = lax.axis_index("s")

  zv = jnp.zeros((L,), jnp.float32)

  def zbody(i, _):
    cb[pl.ds(i * L, L)] = zv
    return 0

  lax.fori_loop(0, EP // NS // L, zbody, 0)
  pltpu.sync_copy(cb, sh_cnt.at[pl.ds(sid * (EP // NS), EP // NS)])

  ov = jnp.ones((L,), jnp.float32)
  for j in range(BATCH // L):
    ones_v[pl.ds(j * L, L)] = ov

  pltpu.sync_copy(segs.at[cid, sid], ids_v)
  plsc.subcore_barrier()

  def sbody(j, _):
    pltpu.sync_copy(ones_v, sh_cnt.at[ids_v.at[j]], add=True)
    return 0

  lax.fori_loop(0, NBATCH, sbody, 0)
  plsc.subcore_barrier()

  def emit(out_ref, per_tile):
    pltpu.sync_copy(sh_cnt.at[pl.ds(sid * per_tile, per_tile)],
                    cb.at[pl.ds(0, per_tile)])

    def rbody(i, _):
      v = cb[pl.ds(i * L, L)]
      cb[pl.ds(i * L, L)] = 1.0 / jnp.maximum(v, 1.0)
      return 0

    lax.fori_loop(0, per_tile // L, rbody, 0)
    pltpu.sync_copy(cb.at[pl.ds(0, per_tile)],
                    out_ref.at[pl.ds(sid * per_tile, per_tile)])

  @pl.when(cid == 0)
  def _():
    emit(recip_e, EP // NS)

  @pl.when(cid == 1)
  def _():
    emit(recip_n, NP // NS)


@functools.partial(
    pl.kernel,
    out_type=(
        jax.ShapeDtypeStruct((EP,), jnp.float32),
        jax.ShapeDtypeStruct((NP,), jnp.float32),
    ),
    mesh=_MESH,
    compiler_params=pltpu.CompilerParams(use_tc_tiling_on_sc=False),
    scratch_types=[
        pltpu.VMEM_SHARED((EP,), jnp.float32),
        pltpu.VMEM((NBATCH, BATCH), jnp.int32),
        pltpu.VMEM((BATCH,), jnp.float32),
        pltpu.VMEM((EP // NS,), jnp.float32),
    ],
)
def _counts(segs, recip_e, recip_n, sh_cnt, ids_v, ones_v, cb):
  _counts_body(segs, recip_e, recip_n, sh_cnt, ids_v, ones_v, cb)


def _mp_body(x2, eb_i, eb_s, nb_i, nb_s, recip_e, recip_n, out2, edge_mid,
             sh_acc, buf_a, idx_v, seg_v, rows0, rows1, rcp_v,
             g0, g1, s0, s1):
  cid = lax.axis_index("c")
  sid = lax.axis_index("s")
  nrow = NP // NS
  erow = EP // NS

  def zero_slab():
    zv = jnp.zeros((L,), jnp.float32)

    def body(r, _):
      for j in range(C // L):
        buf_a[r, pl.ds(j * L, L)] = zv
      return 0

    lax.fori_loop(0, SLAB, body, 0)

  def zero_shared(base, rows):
    def body(k, _):
      pltpu.sync_copy(buf_a, sh_acc.at[pl.ds(base + k * SLAB, SLAB), :])
      return 0

    lax.fori_loop(0, rows // SLAB, body, 0)

  def gather_scatter(src_hbm, idx_arr, seg_arr, acc_rows):
    acc = sh_acc.at[pl.ds(0, acc_rows), :]
    for half in range(2):
      pltpu.sync_copy(idx_arr.at[sid, pl.ds(half * NHALF, NHALF)], idx_v)
      pltpu.sync_copy(seg_arr.at[sid, pl.ds(half * NHALF, NHALF)], seg_v)
      pltpu.async_copy(src_hbm.at[idx_v.at[0]], rows0, g0)
      pltpu.async_copy(src_hbm.at[idx_v.at[1]], rows1, g1)

      def pair(p, _):
        j = 2 * p
        pltpu.make_async_copy(src_hbm.at[idx_v.at[0]], rows0, g0).wait()
        sc0 = pltpu.async_copy(rows0, acc.at[seg_v.at[j]], s0, add=True)
        pltpu.make_async_copy(src_hbm.at[idx_v.at[1]], rows1, g1).wait()
        sc1 = pltpu.async_copy(rows1, acc.at[seg_v.at[j + 1]], s1, add=True)
        sc0.wait()

        @pl.when(p < NHALF // 2 - 1)
        def _():
          pltpu.async_copy(src_hbm.at[idx_v.at[j + 2]], rows0, g0)

        sc1.wait()

        @pl.when(p < NHALF // 2 - 1)
        def _():
          pltpu.async_copy(src_hbm.at[idx_v.at[j + 3]], rows1, g1)

        return 0

      lax.fori_loop(0, NHALF // 2, pair, 0)

  def normalize_emit(n_slabs, rcp_base, dst_at):
    tbase = sid * n_slabs * SLAB

    def body(k, _):
      pltpu.sync_copy(sh_acc.at[pl.ds(tbase + k * SLAB, SLAB), :], buf_a)

      def rows(i, _):
        rv = rcp_v[pl.ds(rcp_base + k * SLAB + i * L, L)]
        for t in range(L):
          r = i * L + t
          s = rv[t]
          for j in range(C // L):
            v = buf_a[r, pl.ds(j * L, L)]
            buf_a[r, pl.ds(j * L, L)] = jnp.maximum(v * s, 0.0)
        return 0

      lax.fori_loop(0, SLAB // L, rows, 0)
      pltpu.sync_copy(buf_a, dst_at(tbase + k * SLAB))
      return 0

    lax.fori_loop(0, n_slabs, body, 0)

  zero_slab()
  zero_shared(sid * erow, erow)
  plsc.subcore_barrier()

  @pl.when(cid == 0)
  def _():
    gather_scatter(x2.at[0], eb_i, eb_s, EP)

  @pl.when(cid == 1)
  def _():
    gather_scatter(x2.at[1], eb_i, eb_s, EP)

  plsc.subcore_barrier()

  pltpu.sync_copy(recip_e.at[pl.ds(sid * erow, erow)], rcp_v)
  normalize_emit(erow // SLAB, 0,
                 lambda b: edge_mid.at[cid, pl.ds(b, SLAB), :])
  plsc.subcore_barrier()

  zero_slab()
  zero_shared(sid * nrow, nrow)
  plsc.subcore_barrier()

  @pl.when(cid == 0)
  def _():
    gather_scatter(edge_mid.at[0], nb_i, nb_s, NP)

  @pl.when(cid == 1)
  def _():
    gather_scatter(edge_mid.at[1], nb_i, nb_s, NP)

  plsc.subcore_barrier()

  pltpu.sync_copy(recip_n.at[pl.ds(sid * nrow, nrow)],
                  rcp_v.at[pl.ds(0, nrow)])
  normalize_emit(nrow // SLAB, 0,
                 lambda b: out2.at[cid, pl.ds(b, SLAB), :])


@functools.partial(
    pl.kernel,
    out_type=(
        jax.ShapeDtypeStruct((NC, NP, C), jnp.float32),
        jax.ShapeDtypeStruct((NC, EP, C), jnp.float32),
    ),
    mesh=_MESH,
    compiler_params=pltpu.CompilerParams(use_tc_tiling_on_sc=False),
    scratch_types=[
        pltpu.VMEM_SHARED((EP, C), jnp.float32),
        pltpu.VMEM((SLAB, C), jnp.float32),
        pltpu.VMEM((NHALF, BATCH), jnp.int32),
        pltpu.VMEM((NHALF, BATCH), jnp.int32),
        pltpu.VMEM((BATCH, C), jnp.float32),
        pltpu.VMEM((BATCH, C), jnp.float32),
        pltpu.VMEM((EP // NS,), jnp.float32),
        pltpu.SemaphoreType.DMA,
        pltpu.SemaphoreType.DMA,
        pltpu.SemaphoreType.DMA,
        pltpu.SemaphoreType.DMA,
    ],
)
def _mp(x2, eb_i, eb_s, nb_i, nb_s, recip_e, recip_n, out2, edge_mid,
        sh_acc, buf_a, idx_v, seg_v, rows0, rows1, rcp_v, g0, g1, s0, s1):
  _mp_body(x2, eb_i, eb_s, nb_i, nb_s, recip_e, recip_n, out2, edge_mid,
           sh_acc, buf_a, idx_v, seg_v, rows0, rows1, rcp_v, g0, g1, s0, s1)


def _mlp_kernel(x_ref, w1_ref, b1_ref, w2_ref, b2_ref, o_ref):
  h = jnp.maximum(
      jnp.dot(x_ref[...], w1_ref[...], preferred_element_type=jnp.float32)
      + b1_ref[...], 0.0)
  logits = jnp.dot(h, w2_ref[...], preferred_element_type=jnp.float32) + b2_ref[...]
  m = jnp.max(logits, axis=-1, keepdims=True)
  lse = jnp.log(jnp.sum(jnp.exp(logits - m), axis=-1, keepdims=True)) + m
  o_ref[...] = logits - lse


def _mlp(x, w1, b1, w2, b2):
  rows, blk = x.shape[0], 1000
  d_in, d_h = w1.shape
  d_out = w2.shape[1]
  return pl.pallas_call(
      _mlp_kernel,
      grid=(rows // blk,),
      in_specs=[
          pl.BlockSpec((blk, d_in), lambda i: (i, 0)),
          pl.BlockSpec((d_in, d_h), lambda i: (0, 0)),
          pl.BlockSpec((1, d_h), lambda i: (0, 0)),
          pl.BlockSpec((d_h, d_out), lambda i: (0, 0)),
          pl.BlockSpec((1, d_out), lambda i: (0, 0)),
      ],
      out_specs=pl.BlockSpec((blk, d_out), lambda i: (i, 0)),
      out_shape=jax.ShapeDtypeStruct((rows, d_out), jnp.float32),
  )(x, w1, b1.reshape(1, -1), w2, b2.reshape(1, -1))


def kernel(node_x, eb_nodes_map, eb_batch, nb_edges_map, nb_batch,
           W1, b1, W2, b2):
  i32 = jnp.int32
  pad = MP - N_MEMB
  shape3 = (NS, NBATCH, BATCH)
  eb_i = jnp.concatenate([eb_nodes_map.astype(i32),
                          jnp.zeros((pad,), i32)]).reshape(shape3)
  eb_s = jnp.concatenate([eb_batch.astype(i32),
                          jnp.full((pad,), N_EDGES, i32)]).reshape(shape3)
  nb_i = jnp.concatenate([nb_edges_map.astype(i32),
                          jnp.zeros((pad,), i32)]).reshape(shape3)
  nb_s = jnp.concatenate([nb_batch.astype(i32),
                          jnp.full((pad,), N_NODES, i32)]).reshape(shape3)
  segs = jnp.stack([eb_s, nb_s])

  recip_e, recip_n = _counts(segs)

  x2 = jnp.pad(node_x, ((0, NP - N_NODES), (0, 0)))
  x2 = x2.reshape(NP, NC, C).transpose(1, 0, 2)

  n1_2, _ = _mp(x2, eb_i, eb_s, nb_i, nb_s, recip_e, recip_n)
  m_2, _ = _mp(n1_2, eb_i, eb_s, nb_i, nb_s, recip_e, recip_n)

  n1 = jnp.concatenate([n1_2[0, :N_NODES], n1_2[1, :N_NODES]], axis=1)
  mm = jnp.concatenate([m_2[0, :N_NODES], m_2[1, :N_NODES]], axis=1)
  x_final = jnp.concatenate([node_x, n1, mm], axis=1)

  w1_eff = jnp.concatenate(
      [W1[:D_FEAT], W1[D_FEAT:2 * D_FEAT] + W1[2 * D_FEAT:3 * D_FEAT],
       W1[3 * D_FEAT:]], axis=0)

  return _mlp(x_final, w1_eff, b1, W2, b2)
